# Initial kernel scaffold; baseline (speedup 1.0000x reference)
import jax
import jax.numpy as jnp
from jax.experimental import pallas as pl


def kernel(f2, f3, f4, w2, b2, w3, b3, w4, b4):
    raise NotImplementedError("write your pallas kernel here")


# speedup vs baseline: 1.0528x; 1.0528x over previous
import functools

import jax
import jax.numpy as jnp
from jax.experimental import pallas as pl
from jax.experimental.pallas import tpu as pltpu

_EMBED = 512
_K = 8
_G = 4


def _up_matrix(hin, hout):
    eye = jnp.eye(hin, dtype=jnp.float32)
    return jax.image.resize(eye, (hout, hin), method="linear")


def _body(f2_ref, f3_ref, f4_ref, w2_ref, w3_ref, w4_ref, m3_ref, m4_ref,
          bb_ref, g_ref, tok_ref, *, hw):
    f32 = jnp.float32
    hi = jax.lax.Precision.HIGHEST
    dn_std = (((1,), (0,)), ((), ()))
    dn_nt = (((1,), (1,)), ((), ()))
    lanes = jax.lax.broadcasted_iota(jnp.int32, (1, hw), 1)
    sub8 = jax.lax.broadcasted_iota(jnp.int32, (8, hw), 0)
    crow = jnp.where(sub8 == 0, f32(1.0 / hw), f32(0.0))

    bf = jnp.bfloat16
    w2b, w3b, w4b = w2_ref[...], w3_ref[...], w4_ref[...]
    for g in range(_G):
        p2 = jax.lax.dot_general(w2b, f2_ref[g].astype(bf), dn_std,
                                 preferred_element_type=f32)
        p3 = jax.lax.dot_general(w3b, f3_ref[g].astype(bf), dn_std,
                                 preferred_element_type=f32)
        u3 = jax.lax.dot_general(p3, m3_ref[...], dn_std,
                                 precision=hi, preferred_element_type=f32)
        p4 = jax.lax.dot_general(w4b, f4_ref[g].astype(bf), dn_std,
                                 preferred_element_type=f32)
        u4 = jax.lax.dot_general(p4, m4_ref[...], dn_std,
                                 precision=hi, preferred_element_type=f32)
        fmap = u4 + u3 + p2 + bb_ref[...]

        heat = jnp.sum(jnp.abs(fmap), axis=0, keepdims=True)
        masks = []
        h = heat
        for _ in range(_K):
            idx = jnp.argmax(h, axis=1, keepdims=True)
            sel = lanes == idx
            masks.append(jnp.where(sel, f32(1.0), f32(0.0)))
            h = jnp.where(sel, f32(-1.0), h)
        oh = jnp.concatenate(masks + [crow], axis=0)

        rt = jax.lax.dot_general(fmap, oh, dn_nt,
                                 precision=hi, preferred_element_type=f32)
        res = rt.T
        tok_ref[g] = res[0:_K, :]
        g_ref[g] = res[_K:_K + 1, :]


def kernel(f2, f3, f4, w2, b2, w3, b3, w4, b4):
    B, T = f2.shape[:2]
    N = B * T
    H, W = f2.shape[-2:]
    hw = H * W
    c2, c3, c4 = f2.shape[2], f3.shape[2], f4.shape[2]
    hw3 = f3.shape[-2] * f3.shape[-1]
    hw4 = f4.shape[-2] * f4.shape[-1]

    f2r = f2.reshape(N, c2, hw)
    f3r = f3.reshape(N, c3, hw3)
    f4r = f4.reshape(N, c4, hw4)

    r3 = _up_matrix(f3.shape[-2], H)
    m3t = jnp.kron(r3, r3).T
    r4 = _up_matrix(f4.shape[-2], H)
    m4t = jnp.kron(r4, r4).T
    bb = jnp.broadcast_to((b2 + b3 + b4)[:, None], (_EMBED, hw))

    grid = (N // _G,)
    const = lambda n: (0, 0)
    g_out, tok_out = pl.pallas_call(
        functools.partial(_body, hw=hw),
        grid=grid,
        in_specs=[
            pl.BlockSpec((_G, c2, hw), lambda n: (n, 0, 0)),
            pl.BlockSpec((_G, c3, hw3), lambda n: (n, 0, 0)),
            pl.BlockSpec((_G, c4, hw4), lambda n: (n, 0, 0)),
            pl.BlockSpec((_EMBED, c2), const),
            pl.BlockSpec((_EMBED, c3), const),
            pl.BlockSpec((_EMBED, c4), const),
            pl.BlockSpec((hw3, hw), const),
            pl.BlockSpec((hw4, hw), const),
            pl.BlockSpec((_EMBED, hw), const),
        ],
        out_specs=[
            pl.BlockSpec((_G, 1, _EMBED), lambda n: (n, 0, 0)),
            pl.BlockSpec((_G, _K, _EMBED), lambda n: (n, 0, 0)),
        ],
        out_shape=[
            jax.ShapeDtypeStruct((N, 1, _EMBED), jnp.float32),
            jax.ShapeDtypeStruct((N, _K, _EMBED), jnp.float32),
        ],
        compiler_params=pltpu.CompilerParams(
            dimension_semantics=("parallel",),
            vmem_limit_bytes=56 * 1024 * 1024,
        ),
    )(f2r, f3r, f4r, w2.astype(jnp.bfloat16), w3.astype(jnp.bfloat16),
      w4.astype(jnp.bfloat16), m3t, m4t, bb)

    return g_out.reshape(B, T, _EMBED), tok_out.reshape(B, T, _K, _EMBED)


# speedup vs baseline: 1.1645x; 1.1061x over previous
import functools

import jax
import jax.numpy as jnp
from jax.experimental import pallas as pl
from jax.experimental.pallas import tpu as pltpu

_EMBED = 512
_K = 8
_G = 4


def _up_matrix(hin, hout):
    eye = jnp.eye(hin, dtype=jnp.float32)
    return jax.image.resize(eye, (hout, hin), method="linear")


def _body(f2_ref, f3_ref, f4_ref, w2_ref, w3_ref, w4_ref, m3_ref, m4_ref,
          bb_ref, g_ref, tok_ref, *, hw):
    f32 = jnp.float32
    hi = jax.lax.Precision.HIGHEST
    dn_std = (((1,), (0,)), ((), ()))
    dn_nt = (((1,), (1,)), ((), ()))
    lanes = jax.lax.broadcasted_iota(jnp.int32, (1, hw), 1)
    sub8 = jax.lax.broadcasted_iota(jnp.int32, (8, hw), 0)
    crow = jnp.where(sub8 == 0, f32(1.0 / hw), f32(0.0))

    bf = jnp.bfloat16
    w2b, w3b, w4b = w2_ref[...], w3_ref[...], w4_ref[...]
    for g in range(_G):
        p2 = jax.lax.dot_general(w2b, f2_ref[g], dn_std,
                                 preferred_element_type=f32)
        p3 = jax.lax.dot_general(w3b, f3_ref[g], dn_std,
                                 preferred_element_type=f32)
        u3 = jax.lax.dot_general(p3, m3_ref[...], dn_std,
                                 precision=hi, preferred_element_type=f32)
        p4 = jax.lax.dot_general(w4b, f4_ref[g], dn_std,
                                 preferred_element_type=f32)
        u4 = jax.lax.dot_general(p4, m4_ref[...], dn_std,
                                 precision=hi, preferred_element_type=f32)
        fmap = u4 + u3 + p2 + bb_ref[...]

        heat = jnp.sum(jnp.abs(fmap), axis=0, keepdims=True)
        masks = []
        h = heat
        for _ in range(_K):
            idx = jnp.argmax(h, axis=1, keepdims=True)
            sel = lanes == idx
            masks.append(jnp.where(sel, f32(1.0), f32(0.0)))
            h = jnp.where(sel, f32(-1.0), h)
        oh = jnp.concatenate(masks + [crow], axis=0)

        rt = jax.lax.dot_general(fmap, oh, dn_nt,
                                 precision=hi, preferred_element_type=f32)
        res = rt.T
        tok_ref[g] = res[0:_K, :]
        g_ref[g] = res[_K:_K + 1, :]


def kernel(f2, f3, f4, w2, b2, w3, b3, w4, b4):
    B, T = f2.shape[:2]
    N = B * T
    H, W = f2.shape[-2:]
    hw = H * W
    c2, c3, c4 = f2.shape[2], f3.shape[2], f4.shape[2]
    hw3 = f3.shape[-2] * f3.shape[-1]
    hw4 = f4.shape[-2] * f4.shape[-1]

    f2r = f2.reshape(N, c2, hw).astype(jnp.bfloat16)
    f3r = f3.reshape(N, c3, hw3).astype(jnp.bfloat16)
    f4r = f4.reshape(N, c4, hw4).astype(jnp.bfloat16)

    r3 = _up_matrix(f3.shape[-2], H)
    m3t = jnp.kron(r3, r3).T
    r4 = _up_matrix(f4.shape[-2], H)
    m4t = jnp.kron(r4, r4).T
    bb = jnp.broadcast_to((b2 + b3 + b4)[:, None], (_EMBED, hw))

    grid = (N // _G,)
    const = lambda n: (0, 0)
    g_out, tok_out = pl.pallas_call(
        functools.partial(_body, hw=hw),
        grid=grid,
        in_specs=[
            pl.BlockSpec((_G, c2, hw), lambda n: (n, 0, 0)),
            pl.BlockSpec((_G, c3, hw3), lambda n: (n, 0, 0)),
            pl.BlockSpec((_G, c4, hw4), lambda n: (n, 0, 0)),
            pl.BlockSpec((_EMBED, c2), const),
            pl.BlockSpec((_EMBED, c3), const),
            pl.BlockSpec((_EMBED, c4), const),
            pl.BlockSpec((hw3, hw), const),
            pl.BlockSpec((hw4, hw), const),
            pl.BlockSpec((_EMBED, hw), const),
        ],
        out_specs=[
            pl.BlockSpec((_G, 1, _EMBED), lambda n: (n, 0, 0)),
            pl.BlockSpec((_G, _K, _EMBED), lambda n: (n, 0, 0)),
        ],
        out_shape=[
            jax.ShapeDtypeStruct((N, 1, _EMBED), jnp.float32),
            jax.ShapeDtypeStruct((N, _K, _EMBED), jnp.float32),
        ],
        compiler_params=pltpu.CompilerParams(
            dimension_semantics=("parallel",),
            vmem_limit_bytes=56 * 1024 * 1024,
        ),
    )(f2r, f3r, f4r, w2.astype(jnp.bfloat16), w3.astype(jnp.bfloat16),
      w4.astype(jnp.bfloat16), m3t, m4t, bb)

    return g_out.reshape(B, T, _EMBED), tok_out.reshape(B, T, _K, _EMBED)


# speedup vs baseline: 1.1664x; 1.0016x over previous
import functools

import jax
import jax.numpy as jnp
from jax.experimental import pallas as pl
from jax.experimental.pallas import tpu as pltpu

_EMBED = 512
_K = 8
_G = 4


def _up_matrix(hin, hout):
    eye = jnp.eye(hin, dtype=jnp.float32)
    return jax.image.resize(eye, (hout, hin), method="linear")


def _body(f2_ref, f3_ref, f4_ref, w2_ref, w3_ref, w4_ref, m3_ref, m4_ref,
          bb_ref, g_ref, tok_ref, *, hw):
    f32 = jnp.float32
    hi = jax.lax.Precision.HIGHEST
    dn_std = (((1,), (0,)), ((), ()))
    dn_nt = (((1,), (1,)), ((), ()))
    lanes = jax.lax.broadcasted_iota(jnp.int32, (1, hw), 1)
    sub8 = jax.lax.broadcasted_iota(jnp.int32, (8, hw), 0)
    crow = jnp.where(sub8 == 0, f32(1.0 / hw), f32(0.0))

    w2b, w3b, w4b = w2_ref[...], w3_ref[...], w4_ref[...]
    for g in range(_G):
        p2 = jax.lax.dot_general(w2b, f2_ref[g], dn_std,
                                 preferred_element_type=f32)
        p3 = jax.lax.dot_general(w3b, f3_ref[g], dn_std,
                                 preferred_element_type=f32)
        u3 = jax.lax.dot_general(p3, m3_ref[...], dn_std,
                                 precision=hi, preferred_element_type=f32)
        p4 = jax.lax.dot_general(w4b, f4_ref[g], dn_std,
                                 preferred_element_type=f32)
        u4 = jax.lax.dot_general(p4, m4_ref[...], dn_std,
                                 precision=hi, preferred_element_type=f32)
        fmap = u4 + u3 + p2 + bb_ref[...]

        heat = jnp.sum(jnp.abs(fmap), axis=0, keepdims=True)
        masks = []
        h = heat
        for _ in range(_K):
            idx = jnp.argmax(h, axis=1, keepdims=True)
            sel = lanes == idx
            masks.append(jnp.where(sel, f32(1.0), f32(0.0)))
            h = jnp.where(sel, f32(-1.0), h)
        oh = jnp.concatenate(masks + [crow], axis=0)

        rt = jax.lax.dot_general(fmap, oh, dn_nt,
                                 precision=hi, preferred_element_type=f32)
        res = rt.T
        tok_ref[g] = res[0:_K, :]
        g_ref[g] = res[_K:_K + 1, :]


def kernel(f2, f3, f4, w2, b2, w3, b3, w4, b4):
    B, T = f2.shape[:2]
    N = B * T
    H, W = f2.shape[-2:]
    hw = H * W
    c2, c3, c4 = f2.shape[2], f3.shape[2], f4.shape[2]
    hw3 = f3.shape[-2] * f3.shape[-1]
    hw4 = f4.shape[-2] * f4.shape[-1]

    f2r = f2.reshape(N, c2, hw).astype(jnp.bfloat16)
    f3r = f3.reshape(N, c3, hw3).astype(jnp.bfloat16)
    f4r = f4.reshape(N, c4, hw4).astype(jnp.bfloat16)

    r3 = _up_matrix(f3.shape[-2], H)
    m3t = jnp.kron(r3, r3).T
    r4 = _up_matrix(f4.shape[-2], H)
    m4t = jnp.kron(r4, r4).T
    bb = jnp.broadcast_to((b2 + b3 + b4)[:, None], (_EMBED, hw))

    grid = (N // _G,)
    const = lambda n: (0, 0)
    g_out, tok_out = pl.pallas_call(
        functools.partial(_body, hw=hw),
        grid=grid,
        in_specs=[
            pl.BlockSpec((_G, c2, hw), lambda n: (n, 0, 0)),
            pl.BlockSpec((_G, c3, hw3), lambda n: (n, 0, 0)),
            pl.BlockSpec((_G, c4, hw4), lambda n: (n, 0, 0)),
            pl.BlockSpec((_EMBED, c2), const),
            pl.BlockSpec((_EMBED, c3), const),
            pl.BlockSpec((_EMBED, c4), const),
            pl.BlockSpec((hw3, hw), const),
            pl.BlockSpec((hw4, hw), const),
            pl.BlockSpec((_EMBED, hw), const),
        ],
        out_specs=[
            pl.BlockSpec((_G, 1, _EMBED), lambda n: (n, 0, 0)),
            pl.BlockSpec((_G, _K, _EMBED), lambda n: (n, 0, 0)),
        ],
        out_shape=[
            jax.ShapeDtypeStruct((N, 1, _EMBED), jnp.float32),
            jax.ShapeDtypeStruct((N, _K, _EMBED), jnp.float32),
        ],
        compiler_params=pltpu.CompilerParams(
            dimension_semantics=("parallel",),
            vmem_limit_bytes=56 * 1024 * 1024,
        ),
    )(f2r, f3r, f4r, w2.astype(jnp.bfloat16), w3.astype(jnp.bfloat16),
      w4.astype(jnp.bfloat16), m3t, m4t, bb)

    return g_out.reshape(B, T, _EMBED), tok_out.reshape(B, T, _K, _EMBED)
